# Initial kernel scaffold; baseline (speedup 1.0000x reference)
import jax
import jax.numpy as jnp
from jax.experimental import pallas as pl


def kernel(x):
    raise NotImplementedError("write your pallas kernel here")


# speedup vs baseline: 201.6139x; 201.6139x over previous
import functools

import jax
import jax.numpy as jnp
from jax.experimental import pallas as pl
from jax.experimental.pallas import tpu as pltpu

T = 16
SBLK = 64


def _onehot_body(x_ref, o_ref):
    x = x_ref[0]
    st = (jax.nn.sigmoid(x) * (T - 1)).astype(jnp.int32)
    t_iota = jax.lax.broadcasted_iota(jnp.int32, (T,) + st.shape, 0)
    o_ref[0] = (st[None] == t_iota).astype(jnp.float32)


def kernel(x):
    B, S, D = x.shape
    grid = (B, S // SBLK)
    return pl.pallas_call(
        _onehot_body,
        grid=grid,
        in_specs=[pl.BlockSpec((1, SBLK, D), lambda b, s: (b, s, 0))],
        out_specs=pl.BlockSpec((1, T, SBLK, D), lambda b, s: (b, 0, s, 0)),
        out_shape=jax.ShapeDtypeStruct((B, T, S, D), jnp.float32),
    )(x)
